# baseline (speedup 1.0000x reference)
import functools

import jax
import jax.numpy as jnp
from jax import lax
from jax.experimental import pallas as pl
from jax.experimental.pallas import tpu as pltpu
from jax.experimental.pallas import tpu_sc as plsc

_NC = 2
_NS = 16
_NW = _NC * _NS
_CHUNK = 128

_D = 128
_N_PAD = 100096

_GEOM = {1: (100000, 106496, 26), 2: (150000, 155648, 38), 3: (70000, 73728, 18)}


def _mish(x):
    return x * jnp.tanh(jax.nn.softplus(x))


def _sc_gather(emb, idx_pad, n_chunks):
    e_pad = idx_pad.shape[0]
    assert e_pad == _NW * n_chunks * _CHUNK
    mesh = plsc.VectorSubcoreMesh(core_axis_name="c", subcore_axis_name="s")

    @functools.partial(
        pl.kernel,
        out_type=jax.ShapeDtypeStruct((e_pad, _D), jnp.float32),
        mesh=mesh,
        scratch_types=[
            pltpu.VMEM((_CHUNK,), jnp.int32),
            pltpu.VMEM((_CHUNK,), jnp.int32),
            pltpu.VMEM((_CHUNK, _D), jnp.float32),
            pltpu.VMEM((_CHUNK, _D), jnp.float32),
            pltpu.SemaphoreType.DMA,
            pltpu.SemaphoreType.DMA,
        ],
    )
    def gather_kernel(emb_hbm, idx_hbm, out_hbm, idx0, idx1, rows0, rows1,
                      sem0, sem1):
        wid = lax.axis_index("s") * _NC + lax.axis_index("c")
        base = wid * (n_chunks * _CHUNK)
        idx_v = (idx0, idx1)
        rows_v = (rows0, rows1)
        sems = (sem0, sem1)

        def start(i, b):
            pltpu.sync_copy(idx_hbm.at[pl.ds(base + i * _CHUNK, _CHUNK)],
                            idx_v[b])
            pltpu.async_copy(emb_hbm.at[idx_v[b]], rows_v[b], sems[b])

        def finish(i, b):
            pltpu.make_async_copy(emb_hbm.at[idx_v[b]], rows_v[b],
                                  sems[b]).wait()
            pltpu.sync_copy(rows_v[b],
                            out_hbm.at[pl.ds(base + i * _CHUNK, _CHUNK)])

        start(0, 0)

        @pl.loop(0, n_chunks // 2)
        def _body(g):
            i = 2 * g
            start(i + 1, 1)
            finish(i, 0)

            @pl.when(g < n_chunks // 2 - 1)
            def _():
                start(i + 2, 0)
            finish(i + 1, 1)

    return gather_kernel(emb, idx_pad)


_CC = 16
_NCHUNK_COL = _D // _CC
_ROWS_PER_TILE = _N_PAD // _NS
_ZROWS = 782


def _sc_scatter(msgs, idx_flat, counts):
    n_str = len(msgs)
    bases = []
    b = 0
    for k in range(n_str):
        bases.append(b)
        b += counts[k] * _NS * _CHUNK
    mesh = plsc.VectorSubcoreMesh(core_axis_name="c", subcore_axis_name="s")

    @functools.partial(
        pl.kernel,
        out_type=jax.ShapeDtypeStruct((_NC, _NCHUNK_COL // _NC, _N_PAD, _CC),
                                      jnp.float32),
        mesh=mesh,
        scratch_types=[
            pltpu.VMEM((_CHUNK,), jnp.int32),
            pltpu.VMEM((_CHUNK,), jnp.int32),
            pltpu.VMEM((_CHUNK, _CC), jnp.float32),
            pltpu.VMEM((_CHUNK, _CC), jnp.float32),
            pltpu.VMEM((_ZROWS, _CC), jnp.float32),
            pltpu.VMEM_SHARED((_N_PAD, _CC), jnp.float32),
            pltpu.SemaphoreType.DMA,
            pltpu.SemaphoreType.DMA,
        ],
        compiler_params=pltpu.CompilerParams(use_tc_tiling_on_sc=False),
    )
    def scatter_kernel(*refs):
        msgs_hbm = refs[:n_str]
        idx_hbm = refs[n_str]
        out_hbm = refs[n_str + 1]
        idx0, idx1, msg0, msg1, zbuf, acc, sem0, sem1 = refs[n_str + 2:]
        idx_b = (idx0, idx1)
        msg_b = (msg0, msg1)
        sems = (sem0, sem1)

        cid = lax.axis_index("c")
        tid = lax.axis_index("s")

        @pl.loop(0, _ZROWS)
        def _z(i):
            zbuf[i, :] = jnp.zeros((_CC,), jnp.float32)

        r0 = tid * _ROWS_PER_TILE
        for p in range(_NCHUNK_COL // _NC):
            c0 = (_NC * p + cid) * _CC

            for z in range(_ROWS_PER_TILE // _ZROWS):
                pltpu.sync_copy(zbuf, acc.at[pl.ds(r0 + z * _ZROWS, _ZROWS)])
            plsc.subcore_barrier()

            for k in range(n_str):
                cnt = counts[k]
                mh = msgs_hbm[k]
                ib = bases[k]

                def load(j, b, _mh=mh, _cnt=cnt, _ib=ib):
                    row0 = (tid * _cnt + j) * _CHUNK
                    pltpu.async_copy(idx_hbm.at[pl.ds(_ib + row0, _CHUNK)],
                                     idx_b[b], sems[b])
                    pltpu.async_copy(
                        _mh.at[pl.ds(row0, _CHUNK), pl.ds(c0, _CC)],
                        msg_b[b], sems[b])

                def scat(j, b, _mh=mh, _cnt=cnt, _ib=ib):
                    row0 = (tid * _cnt + j) * _CHUNK
                    pltpu.make_async_copy(
                        idx_hbm.at[pl.ds(_ib + row0, _CHUNK)],
                        idx_b[b], sems[b]).wait()
                    pltpu.make_async_copy(
                        _mh.at[pl.ds(row0, _CHUNK), pl.ds(c0, _CC)],
                        msg_b[b], sems[b]).wait()
                    pltpu.sync_copy(msg_b[b], acc.at[idx_b[b]], add=True)

                load(0, 0)

                @pl.loop(0, cnt // 2)
                def _body(g, _load=load, _scat=scat, _cnt=cnt):
                    j = 2 * g
                    _load(j + 1, 1)
                    _scat(j, 0)

                    @pl.when(g < _cnt // 2 - 1)
                    def _():
                        _load(j + 2, 0)
                    _scat(j + 1, 1)
            plsc.subcore_barrier()

            pltpu.sync_copy(acc.at[pl.ds(r0, _ROWS_PER_TILE)],
                            out_hbm.at[cid, p, pl.ds(r0, _ROWS_PER_TILE)])
            plsc.subcore_barrier()

    return scatter_kernel(*msgs, idx_flat)


_BLK = 512


def _relation_mlp(x_all, arity, r_pad, wit, bi, wot, bo):
    k = arity * _D
    nblk = r_pad // _BLK

    def body(*refs):
        x_refs = refs[:arity]
        wit_ref, bi_ref, wot_ref, bo_ref = refs[arity:arity + 4]
        o_refs = refs[arity + 4:]
        xs = [r[...] for r in x_refs]
        wit_v = wit_ref[...].astype(jnp.bfloat16)
        h = bi_ref[...].astype(jnp.float32)
        h = h + sum(jnp.dot(xs[s].astype(jnp.bfloat16),
                            wit_v[s * _D:(s + 1) * _D, :],
                            preferred_element_type=jnp.float32)
                    for s in range(arity))
        h = _mish(h).astype(jnp.bfloat16)
        wot_v = wot_ref[...].astype(jnp.bfloat16)
        bo_v = bo_ref[...]
        for s in range(arity):
            o_refs[s][...] = (xs[s]
                              + jnp.dot(h, wot_v[:, s * _D:(s + 1) * _D],
                                        preferred_element_type=jnp.float32)
                              + bo_v[:, s * _D:(s + 1) * _D])

    in_specs = [pl.BlockSpec((_BLK, _D), functools.partial(
        lambda i, s: (s * nblk + i, 0), s=s)) for s in range(arity)]
    in_specs += [
        pl.BlockSpec((k, k), lambda i: (0, 0)),
        pl.BlockSpec((1, k), lambda i: (0, 0)),
        pl.BlockSpec((k, k), lambda i: (0, 0)),
        pl.BlockSpec((1, k), lambda i: (0, 0)),
    ]
    return pl.pallas_call(
        body,
        grid=(nblk,),
        in_specs=in_specs,
        out_specs=[pl.BlockSpec((_BLK, _D), lambda i: (i, 0))] * arity,
        out_shape=[jax.ShapeDtypeStruct((r_pad, _D), jnp.float32)] * arity,
    )(*([x_all] * arity), wit, bi, wot, bo)


_UBLK = 1000


def _update_mlp(sum_msg, node_emb, wuin_t, buin, wuout_t, buout):
    n = node_emb.shape[0]
    grid = (n // _UBLK,)

    def body(sm_ref, ne_ref, wi_ref, bi_ref, wo_ref, bo_ref, o_ref):
        wi = wi_ref[...].astype(jnp.bfloat16)
        h = (jnp.dot(sm_ref[...].astype(jnp.bfloat16), wi[:_D, :],
                     preferred_element_type=jnp.float32)
             + jnp.dot(ne_ref[...].astype(jnp.bfloat16), wi[_D:, :],
                       preferred_element_type=jnp.float32)
             + bi_ref[...])
        h = _mish(h).astype(jnp.bfloat16)
        o_ref[...] = jnp.dot(h, wo_ref[...].astype(jnp.bfloat16),
                             preferred_element_type=jnp.float32) + bo_ref[...]

    return pl.pallas_call(
        body,
        grid=grid,
        in_specs=[
            pl.BlockSpec((_UBLK, _D), lambda i: (i, 0)),
            pl.BlockSpec((_UBLK, _D), lambda i: (i, 0)),
            pl.BlockSpec((2 * _D, 2 * _D), lambda i: (0, 0)),
            pl.BlockSpec((1, 2 * _D), lambda i: (0, 0)),
            pl.BlockSpec((2 * _D, _D), lambda i: (0, 0)),
            pl.BlockSpec((1, _D), lambda i: (0, 0)),
        ],
        out_specs=pl.BlockSpec((_UBLK, _D), lambda i: (i, 0)),
        out_shape=jax.ShapeDtypeStruct((n, _D), jnp.float32),
    )(sum_msg, node_emb, wuin_t, buin, wuout_t, buout)


def _slot_idx(idx, arity, r_pad, fill):
    slots = idx.reshape(-1, arity).T
    return jnp.pad(slots, ((0, 0), (0, r_pad - slots.shape[1])),
                   constant_values=fill)


def kernel(node_embeddings, rel_unary_idx, rel_binary_idx, rel_ternary_idx,
           W1_in, b1_in, W1_out, b1_out,
           W2_in, b2_in, W2_out, b2_out,
           W3_in, b3_in, W3_out, b3_out,
           Wu_in, bu_in, Wu_out, bu_out):
    n = node_embeddings.shape[0]
    idxs = {1: rel_unary_idx, 2: rel_binary_idx, 3: rel_ternary_idx}
    weights = {1: (W1_in, b1_in, W1_out, b1_out),
               2: (W2_in, b2_in, W2_out, b2_out),
               3: (W3_in, b3_in, W3_out, b3_out)}

    gathered = {}
    for a in (1, 2, 3):
        _, r_pad, cnt = _GEOM[a]
        idx_g = _slot_idx(idxs[a], a, r_pad, 0)
        gathered[a] = _sc_gather(node_embeddings, idx_g.reshape(-1), a * cnt)

    msgs, idx_parts, counts = [], [], []
    for a in (1, 2, 3):
        _, r_pad, cnt = _GEOM[a]
        idx_s = _slot_idx(idxs[a], a, r_pad, n)
        wi, bi, wo, bo = weights[a]
        ms = _relation_mlp(gathered[a], a, r_pad, wi.T, bi.reshape(1, -1),
                           wo.T, bo.reshape(1, -1))
        ms = ms if isinstance(ms, (list, tuple)) else [ms]
        for s in range(a):
            msgs.append(ms[s])
            idx_parts.append(idx_s[s])
            counts.append(r_pad // (_NS * _CHUNK))

    pieces = _sc_scatter(msgs, jnp.concatenate(idx_parts), tuple(counts))
    sum_msg = pieces.transpose(2, 1, 0, 3).reshape(_N_PAD, _D)

    return _update_mlp(sum_msg, node_embeddings,
                       Wu_in.T, bu_in.reshape(1, -1),
                       Wu_out.T, bu_out.reshape(1, -1))


# speedup vs baseline: 1.0032x; 1.0032x over previous
import functools

import jax
import jax.numpy as jnp
from jax import lax
from jax.experimental import pallas as pl
from jax.experimental.pallas import tpu as pltpu
from jax.experimental.pallas import tpu_sc as plsc

_NC = 2
_NS = 16
_NW = _NC * _NS
_CHUNK = 128

_D = 128
_N_PAD = 100096

_GEOM = {1: (100000, 106496, 26), 2: (150000, 155648, 38), 3: (70000, 73728, 18)}


def _mish(x):
    return x * jnp.tanh(jax.nn.softplus(x))


def _sc_gather(emb, idx_pad, n_chunks):
    e_pad = idx_pad.shape[0]
    assert e_pad == _NW * n_chunks * _CHUNK
    mesh = plsc.VectorSubcoreMesh(core_axis_name="c", subcore_axis_name="s")

    @functools.partial(
        pl.kernel,
        out_type=jax.ShapeDtypeStruct((e_pad, _D), jnp.float32),
        mesh=mesh,
        scratch_types=[
            pltpu.VMEM((n_chunks, _CHUNK), jnp.int32),
            pltpu.VMEM((_CHUNK, _D), jnp.float32),
            pltpu.VMEM((_CHUNK, _D), jnp.float32),
            pltpu.SemaphoreType.DMA,
            pltpu.SemaphoreType.DMA,
        ],
        compiler_params=pltpu.CompilerParams(use_tc_tiling_on_sc=False),
    )
    def gather_kernel(emb_hbm, idx_hbm, out_hbm, idx_all, rows0, rows1,
                      sem0, sem1):
        wid = lax.axis_index("s") * _NC + lax.axis_index("c")
        base = wid * (n_chunks * _CHUNK)
        rows_v = (rows0, rows1)
        sems = (sem0, sem1)

        pltpu.sync_copy(idx_hbm.at[pl.ds(wid * n_chunks, n_chunks)], idx_all)

        def start(i, b):
            pltpu.async_copy(emb_hbm.at[idx_all.at[i]], rows_v[b], sems[b])

        def finish(i, b):
            pltpu.make_async_copy(emb_hbm.at[idx_all.at[i]], rows_v[b],
                                  sems[b]).wait()
            pltpu.sync_copy(rows_v[b],
                            out_hbm.at[pl.ds(base + i * _CHUNK, _CHUNK)])

        start(0, 0)

        @pl.loop(0, n_chunks // 2)
        def _body(g):
            i = 2 * g
            start(i + 1, 1)
            finish(i, 0)

            @pl.when(g < n_chunks // 2 - 1)
            def _():
                start(i + 2, 0)
            finish(i + 1, 1)

    return gather_kernel(emb, idx_pad.reshape(-1, _CHUNK))


_CC = 16
_NCHUNK_COL = _D // _CC
_ROWS_PER_TILE = _N_PAD // _NS
_ZROWS = 782


def _sc_scatter(msgs, idx_flat, counts):
    n_str = len(msgs)
    bases = []
    b = 0
    for k in range(n_str):
        bases.append(b)
        b += counts[k] * _NS * _CHUNK
    mesh = plsc.VectorSubcoreMesh(core_axis_name="c", subcore_axis_name="s")

    @functools.partial(
        pl.kernel,
        out_type=jax.ShapeDtypeStruct((_NC, _NCHUNK_COL // _NC, _N_PAD, _CC),
                                      jnp.float32),
        mesh=mesh,
        scratch_types=[
            pltpu.VMEM((_CHUNK,), jnp.int32),
            pltpu.VMEM((_CHUNK,), jnp.int32),
            pltpu.VMEM((_CHUNK, _CC), jnp.float32),
            pltpu.VMEM((_CHUNK, _CC), jnp.float32),
            pltpu.VMEM((_ZROWS, _CC), jnp.float32),
            pltpu.VMEM_SHARED((_N_PAD, _CC), jnp.float32),
            pltpu.SemaphoreType.DMA,
            pltpu.SemaphoreType.DMA,
        ],
        compiler_params=pltpu.CompilerParams(use_tc_tiling_on_sc=False),
    )
    def scatter_kernel(*refs):
        msgs_hbm = refs[:n_str]
        idx_hbm = refs[n_str]
        out_hbm = refs[n_str + 1]
        idx0, idx1, msg0, msg1, zbuf, acc, sem0, sem1 = refs[n_str + 2:]
        idx_b = (idx0, idx1)
        msg_b = (msg0, msg1)
        sems = (sem0, sem1)

        cid = lax.axis_index("c")
        tid = lax.axis_index("s")

        @pl.loop(0, _ZROWS)
        def _z(i):
            zbuf[i, :] = jnp.zeros((_CC,), jnp.float32)

        r0 = tid * _ROWS_PER_TILE
        for p in range(_NCHUNK_COL // _NC):
            c0 = (_NC * p + cid) * _CC

            for z in range(_ROWS_PER_TILE // _ZROWS):
                pltpu.sync_copy(zbuf, acc.at[pl.ds(r0 + z * _ZROWS, _ZROWS)])
            plsc.subcore_barrier()

            for k in range(n_str):
                cnt = counts[k]
                mh = msgs_hbm[k]
                ib = bases[k]

                def load(j, b, _mh=mh, _cnt=cnt, _ib=ib):
                    row0 = (tid * _cnt + j) * _CHUNK
                    pltpu.async_copy(idx_hbm.at[pl.ds(_ib + row0, _CHUNK)],
                                     idx_b[b], sems[b])
                    pltpu.async_copy(
                        _mh.at[pl.ds(row0, _CHUNK), pl.ds(c0, _CC)],
                        msg_b[b], sems[b])

                def scat(j, b, _mh=mh, _cnt=cnt, _ib=ib):
                    row0 = (tid * _cnt + j) * _CHUNK
                    pltpu.make_async_copy(
                        idx_hbm.at[pl.ds(_ib + row0, _CHUNK)],
                        idx_b[b], sems[b]).wait()
                    pltpu.make_async_copy(
                        _mh.at[pl.ds(row0, _CHUNK), pl.ds(c0, _CC)],
                        msg_b[b], sems[b]).wait()
                    pltpu.sync_copy(msg_b[b], acc.at[idx_b[b]], add=True)

                load(0, 0)

                @pl.loop(0, cnt // 2)
                def _body(g, _load=load, _scat=scat, _cnt=cnt):
                    j = 2 * g
                    _load(j + 1, 1)
                    _scat(j, 0)

                    @pl.when(g < _cnt // 2 - 1)
                    def _():
                        _load(j + 2, 0)
                    _scat(j + 1, 1)
            plsc.subcore_barrier()

            pltpu.sync_copy(acc.at[pl.ds(r0, _ROWS_PER_TILE)],
                            out_hbm.at[cid, p, pl.ds(r0, _ROWS_PER_TILE)])
            plsc.subcore_barrier()

    return scatter_kernel(*msgs, idx_flat)


_BLK = 512


def _relation_mlp(x_all, arity, r_pad, wit, bi, wot, bo):
    k = arity * _D
    nblk = r_pad // _BLK

    def body(*refs):
        x_refs = refs[:arity]
        wit_ref, bi_ref, wot_ref, bo_ref = refs[arity:arity + 4]
        o_refs = refs[arity + 4:]
        xs = [r[...] for r in x_refs]
        wit_v = wit_ref[...].astype(jnp.bfloat16)
        h = bi_ref[...].astype(jnp.float32)
        h = h + sum(jnp.dot(xs[s].astype(jnp.bfloat16),
                            wit_v[s * _D:(s + 1) * _D, :],
                            preferred_element_type=jnp.float32)
                    for s in range(arity))
        h = _mish(h).astype(jnp.bfloat16)
        wot_v = wot_ref[...].astype(jnp.bfloat16)
        bo_v = bo_ref[...]
        for s in range(arity):
            o_refs[s][...] = (xs[s]
                              + jnp.dot(h, wot_v[:, s * _D:(s + 1) * _D],
                                        preferred_element_type=jnp.float32)
                              + bo_v[:, s * _D:(s + 1) * _D])

    in_specs = [pl.BlockSpec((_BLK, _D), functools.partial(
        lambda i, s: (s * nblk + i, 0), s=s)) for s in range(arity)]
    in_specs += [
        pl.BlockSpec((k, k), lambda i: (0, 0)),
        pl.BlockSpec((1, k), lambda i: (0, 0)),
        pl.BlockSpec((k, k), lambda i: (0, 0)),
        pl.BlockSpec((1, k), lambda i: (0, 0)),
    ]
    return pl.pallas_call(
        body,
        grid=(nblk,),
        in_specs=in_specs,
        out_specs=[pl.BlockSpec((_BLK, _D), lambda i: (i, 0))] * arity,
        out_shape=[jax.ShapeDtypeStruct((r_pad, _D), jnp.float32)] * arity,
    )(*([x_all] * arity), wit, bi, wot, bo)


_UBLK = 1000


def _update_mlp(sum_msg, node_emb, wuin_t, buin, wuout_t, buout):
    n = node_emb.shape[0]
    grid = (n // _UBLK,)

    def body(sm_ref, ne_ref, wi_ref, bi_ref, wo_ref, bo_ref, o_ref):
        wi = wi_ref[...].astype(jnp.bfloat16)
        h = (jnp.dot(sm_ref[...].astype(jnp.bfloat16), wi[:_D, :],
                     preferred_element_type=jnp.float32)
             + jnp.dot(ne_ref[...].astype(jnp.bfloat16), wi[_D:, :],
                       preferred_element_type=jnp.float32)
             + bi_ref[...])
        h = _mish(h).astype(jnp.bfloat16)
        o_ref[...] = jnp.dot(h, wo_ref[...].astype(jnp.bfloat16),
                             preferred_element_type=jnp.float32) + bo_ref[...]

    return pl.pallas_call(
        body,
        grid=grid,
        in_specs=[
            pl.BlockSpec((_UBLK, _D), lambda i: (i, 0)),
            pl.BlockSpec((_UBLK, _D), lambda i: (i, 0)),
            pl.BlockSpec((2 * _D, 2 * _D), lambda i: (0, 0)),
            pl.BlockSpec((1, 2 * _D), lambda i: (0, 0)),
            pl.BlockSpec((2 * _D, _D), lambda i: (0, 0)),
            pl.BlockSpec((1, _D), lambda i: (0, 0)),
        ],
        out_specs=pl.BlockSpec((_UBLK, _D), lambda i: (i, 0)),
        out_shape=jax.ShapeDtypeStruct((n, _D), jnp.float32),
    )(sum_msg, node_emb, wuin_t, buin, wuout_t, buout)


def _slot_idx(idx, arity, r_pad, fill):
    slots = idx.reshape(-1, arity).T
    return jnp.pad(slots, ((0, 0), (0, r_pad - slots.shape[1])),
                   constant_values=fill)


def kernel(node_embeddings, rel_unary_idx, rel_binary_idx, rel_ternary_idx,
           W1_in, b1_in, W1_out, b1_out,
           W2_in, b2_in, W2_out, b2_out,
           W3_in, b3_in, W3_out, b3_out,
           Wu_in, bu_in, Wu_out, bu_out):
    n = node_embeddings.shape[0]
    idxs = {1: rel_unary_idx, 2: rel_binary_idx, 3: rel_ternary_idx}
    weights = {1: (W1_in, b1_in, W1_out, b1_out),
               2: (W2_in, b2_in, W2_out, b2_out),
               3: (W3_in, b3_in, W3_out, b3_out)}

    gathered = {}
    for a in (1, 2, 3):
        _, r_pad, cnt = _GEOM[a]
        idx_g = _slot_idx(idxs[a], a, r_pad, 0)
        gathered[a] = _sc_gather(node_embeddings, idx_g.reshape(-1), a * cnt)

    msgs, idx_parts, counts = [], [], []
    for a in (1, 2, 3):
        _, r_pad, cnt = _GEOM[a]
        idx_s = _slot_idx(idxs[a], a, r_pad, n)
        wi, bi, wo, bo = weights[a]
        ms = _relation_mlp(gathered[a], a, r_pad, wi.T, bi.reshape(1, -1),
                           wo.T, bo.reshape(1, -1))
        ms = ms if isinstance(ms, (list, tuple)) else [ms]
        for s in range(a):
            msgs.append(ms[s])
            idx_parts.append(idx_s[s])
            counts.append(r_pad // (_NS * _CHUNK))

    pieces = _sc_scatter(msgs, jnp.concatenate(idx_parts), tuple(counts))
    sum_msg = pieces.transpose(2, 1, 0, 3).reshape(_N_PAD, _D)

    return _update_mlp(sum_msg, node_embeddings,
                       Wu_in.T, bu_in.reshape(1, -1),
                       Wu_out.T, bu_out.reshape(1, -1))


# speedup vs baseline: 1.0092x; 1.0059x over previous
import functools

import jax
import jax.numpy as jnp
from jax import lax
from jax.experimental import pallas as pl
from jax.experimental.pallas import tpu as pltpu
from jax.experimental.pallas import tpu_sc as plsc

_NC = 2
_NS = 16
_NW = _NC * _NS
_CHUNK = 128

_D = 128
_N_PAD = 100096

_GEOM = {1: (100000, 106496, 26), 2: (150000, 155648, 38), 3: (70000, 73728, 18)}


def _mish(x):
    return x * jnp.tanh(jax.nn.softplus(x))


def _sc_gather(emb, idx_pad, n_chunks):
    e_pad = idx_pad.shape[0]
    assert e_pad == _NW * n_chunks * _CHUNK
    mesh = plsc.VectorSubcoreMesh(core_axis_name="c", subcore_axis_name="s")

    @functools.partial(
        pl.kernel,
        out_type=jax.ShapeDtypeStruct((e_pad, _D), jnp.float32),
        mesh=mesh,
        scratch_types=[
            pltpu.VMEM((n_chunks, _CHUNK), jnp.int32),
            pltpu.VMEM((4, _CHUNK, _D), jnp.float32),
            pltpu.SemaphoreType.DMA((4,)),
            pltpu.SemaphoreType.DMA((4,)),
        ],
        compiler_params=pltpu.CompilerParams(use_tc_tiling_on_sc=False),
    )
    def gather_kernel(emb_hbm, idx_hbm, out_hbm, idx_all, rows_v, gsem, osem):
        wid = lax.axis_index("s") * _NC + lax.axis_index("c")
        base = wid * (n_chunks * _CHUNK)

        pltpu.sync_copy(idx_hbm.at[pl.ds(wid * n_chunks, n_chunks)], idx_all)

        def start(i):
            b = lax.rem(i, 4)
            pltpu.async_copy(emb_hbm.at[idx_all.at[i]], rows_v.at[b],
                             gsem.at[b])

        def finish(i):
            b = lax.rem(i, 4)
            pltpu.make_async_copy(emb_hbm.at[idx_all.at[i]], rows_v.at[b],
                                  gsem.at[b]).wait()
            pltpu.async_copy(rows_v.at[b],
                             out_hbm.at[pl.ds(base + i * _CHUNK, _CHUNK)],
                             osem.at[b])

        def drain(i):
            b = lax.rem(i, 4)
            pltpu.make_async_copy(rows_v.at[b],
                                  out_hbm.at[pl.ds(base + i * _CHUNK, _CHUNK)],
                                  osem.at[b]).wait()

        start(0)
        start(1)
        start(2)

        @pl.loop(0, n_chunks)
        def _body(i):
            finish(i)

            @pl.when(i + 3 < n_chunks)
            def _():
                @pl.when(i >= 1)
                def _():
                    drain(i - 1)
                start(i + 3)

        for t in range(4):
            drain(n_chunks - 4 + t)

    return gather_kernel(emb, idx_pad.reshape(-1, _CHUNK))


_CC = 16
_NCHUNK_COL = _D // _CC
_ROWS_PER_TILE = _N_PAD // _NS
_ZROWS = 782


def _sc_scatter(msgs, idx_flat, counts):
    n_str = len(msgs)
    bases = []
    b = 0
    for k in range(n_str):
        bases.append(b)
        b += counts[k] * _NS * _CHUNK
    mesh = plsc.VectorSubcoreMesh(core_axis_name="c", subcore_axis_name="s")

    @functools.partial(
        pl.kernel,
        out_type=jax.ShapeDtypeStruct((_NC, _NCHUNK_COL // _NC, _N_PAD, _CC),
                                      jnp.float32),
        mesh=mesh,
        scratch_types=[
            pltpu.VMEM((_CHUNK,), jnp.int32),
            pltpu.VMEM((_CHUNK,), jnp.int32),
            pltpu.VMEM((_CHUNK, _CC), jnp.float32),
            pltpu.VMEM((_CHUNK, _CC), jnp.float32),
            pltpu.VMEM((_ZROWS, _CC), jnp.float32),
            pltpu.VMEM_SHARED((_N_PAD, _CC), jnp.float32),
            pltpu.SemaphoreType.DMA,
            pltpu.SemaphoreType.DMA,
        ],
        compiler_params=pltpu.CompilerParams(use_tc_tiling_on_sc=False),
    )
    def scatter_kernel(*refs):
        msgs_hbm = refs[:n_str]
        idx_hbm = refs[n_str]
        out_hbm = refs[n_str + 1]
        idx0, idx1, msg0, msg1, zbuf, acc, sem0, sem1 = refs[n_str + 2:]
        idx_b = (idx0, idx1)
        msg_b = (msg0, msg1)
        sems = (sem0, sem1)

        cid = lax.axis_index("c")
        tid = lax.axis_index("s")

        @pl.loop(0, _ZROWS)
        def _z(i):
            zbuf[i, :] = jnp.zeros((_CC,), jnp.float32)

        r0 = tid * _ROWS_PER_TILE
        for p in range(_NCHUNK_COL // _NC):
            c0 = (_NC * p + cid) * _CC

            for z in range(_ROWS_PER_TILE // _ZROWS):
                pltpu.sync_copy(zbuf, acc.at[pl.ds(r0 + z * _ZROWS, _ZROWS)])
            plsc.subcore_barrier()

            for k in range(n_str):
                cnt = counts[k]
                mh = msgs_hbm[k]
                ib = bases[k]

                def load(j, b, _mh=mh, _cnt=cnt, _ib=ib):
                    row0 = (tid * _cnt + j) * _CHUNK
                    pltpu.async_copy(idx_hbm.at[pl.ds(_ib + row0, _CHUNK)],
                                     idx_b[b], sems[b])
                    pltpu.async_copy(
                        _mh.at[pl.ds(row0, _CHUNK), pl.ds(c0, _CC)],
                        msg_b[b], sems[b])

                def scat(j, b, _mh=mh, _cnt=cnt, _ib=ib):
                    row0 = (tid * _cnt + j) * _CHUNK
                    pltpu.make_async_copy(
                        idx_hbm.at[pl.ds(_ib + row0, _CHUNK)],
                        idx_b[b], sems[b]).wait()
                    pltpu.make_async_copy(
                        _mh.at[pl.ds(row0, _CHUNK), pl.ds(c0, _CC)],
                        msg_b[b], sems[b]).wait()
                    pltpu.sync_copy(msg_b[b], acc.at[idx_b[b]], add=True)

                load(0, 0)

                @pl.loop(0, cnt // 2)
                def _body(g, _load=load, _scat=scat, _cnt=cnt):
                    j = 2 * g
                    _load(j + 1, 1)
                    _scat(j, 0)

                    @pl.when(g < _cnt // 2 - 1)
                    def _():
                        _load(j + 2, 0)
                    _scat(j + 1, 1)
            plsc.subcore_barrier()

            pltpu.sync_copy(acc.at[pl.ds(r0, _ROWS_PER_TILE)],
                            out_hbm.at[cid, p, pl.ds(r0, _ROWS_PER_TILE)])
            plsc.subcore_barrier()

    return scatter_kernel(*msgs, idx_flat)


_BLK = 512


def _relation_mlp(x_all, arity, r_pad, wit, bi, wot, bo):
    k = arity * _D
    nblk = r_pad // _BLK

    def body(*refs):
        x_refs = refs[:arity]
        wit_ref, bi_ref, wot_ref, bo_ref = refs[arity:arity + 4]
        o_refs = refs[arity + 4:]
        xs = [r[...] for r in x_refs]
        wit_v = wit_ref[...].astype(jnp.bfloat16)
        h = bi_ref[...].astype(jnp.float32)
        h = h + sum(jnp.dot(xs[s].astype(jnp.bfloat16),
                            wit_v[s * _D:(s + 1) * _D, :],
                            preferred_element_type=jnp.float32)
                    for s in range(arity))
        h = _mish(h).astype(jnp.bfloat16)
        wot_v = wot_ref[...].astype(jnp.bfloat16)
        bo_v = bo_ref[...]
        for s in range(arity):
            o_refs[s][...] = (xs[s]
                              + jnp.dot(h, wot_v[:, s * _D:(s + 1) * _D],
                                        preferred_element_type=jnp.float32)
                              + bo_v[:, s * _D:(s + 1) * _D])

    in_specs = [pl.BlockSpec((_BLK, _D), functools.partial(
        lambda i, s: (s * nblk + i, 0), s=s)) for s in range(arity)]
    in_specs += [
        pl.BlockSpec((k, k), lambda i: (0, 0)),
        pl.BlockSpec((1, k), lambda i: (0, 0)),
        pl.BlockSpec((k, k), lambda i: (0, 0)),
        pl.BlockSpec((1, k), lambda i: (0, 0)),
    ]
    return pl.pallas_call(
        body,
        grid=(nblk,),
        in_specs=in_specs,
        out_specs=[pl.BlockSpec((_BLK, _D), lambda i: (i, 0))] * arity,
        out_shape=[jax.ShapeDtypeStruct((r_pad, _D), jnp.float32)] * arity,
    )(*([x_all] * arity), wit, bi, wot, bo)


_UBLK = 1000


def _update_mlp(sum_msg, node_emb, wuin_t, buin, wuout_t, buout):
    n = node_emb.shape[0]
    grid = (n // _UBLK,)

    def body(sm_ref, ne_ref, wi_ref, bi_ref, wo_ref, bo_ref, o_ref):
        wi = wi_ref[...].astype(jnp.bfloat16)
        h = (jnp.dot(sm_ref[...].astype(jnp.bfloat16), wi[:_D, :],
                     preferred_element_type=jnp.float32)
             + jnp.dot(ne_ref[...].astype(jnp.bfloat16), wi[_D:, :],
                       preferred_element_type=jnp.float32)
             + bi_ref[...])
        h = _mish(h).astype(jnp.bfloat16)
        o_ref[...] = jnp.dot(h, wo_ref[...].astype(jnp.bfloat16),
                             preferred_element_type=jnp.float32) + bo_ref[...]

    return pl.pallas_call(
        body,
        grid=grid,
        in_specs=[
            pl.BlockSpec((_UBLK, _D), lambda i: (i, 0)),
            pl.BlockSpec((_UBLK, _D), lambda i: (i, 0)),
            pl.BlockSpec((2 * _D, 2 * _D), lambda i: (0, 0)),
            pl.BlockSpec((1, 2 * _D), lambda i: (0, 0)),
            pl.BlockSpec((2 * _D, _D), lambda i: (0, 0)),
            pl.BlockSpec((1, _D), lambda i: (0, 0)),
        ],
        out_specs=pl.BlockSpec((_UBLK, _D), lambda i: (i, 0)),
        out_shape=jax.ShapeDtypeStruct((n, _D), jnp.float32),
    )(sum_msg, node_emb, wuin_t, buin, wuout_t, buout)


def _slot_idx(idx, arity, r_pad, fill):
    slots = idx.reshape(-1, arity).T
    return jnp.pad(slots, ((0, 0), (0, r_pad - slots.shape[1])),
                   constant_values=fill)


def kernel(node_embeddings, rel_unary_idx, rel_binary_idx, rel_ternary_idx,
           W1_in, b1_in, W1_out, b1_out,
           W2_in, b2_in, W2_out, b2_out,
           W3_in, b3_in, W3_out, b3_out,
           Wu_in, bu_in, Wu_out, bu_out):
    n = node_embeddings.shape[0]
    idxs = {1: rel_unary_idx, 2: rel_binary_idx, 3: rel_ternary_idx}
    weights = {1: (W1_in, b1_in, W1_out, b1_out),
               2: (W2_in, b2_in, W2_out, b2_out),
               3: (W3_in, b3_in, W3_out, b3_out)}

    gathered = {}
    for a in (1, 2, 3):
        _, r_pad, cnt = _GEOM[a]
        idx_g = _slot_idx(idxs[a], a, r_pad, 0)
        gathered[a] = _sc_gather(node_embeddings, idx_g.reshape(-1), a * cnt)

    msgs, idx_parts, counts = [], [], []
    for a in (1, 2, 3):
        _, r_pad, cnt = _GEOM[a]
        idx_s = _slot_idx(idxs[a], a, r_pad, n)
        wi, bi, wo, bo = weights[a]
        ms = _relation_mlp(gathered[a], a, r_pad, wi.T, bi.reshape(1, -1),
                           wo.T, bo.reshape(1, -1))
        ms = ms if isinstance(ms, (list, tuple)) else [ms]
        for s in range(a):
            msgs.append(ms[s])
            idx_parts.append(idx_s[s])
            counts.append(r_pad // (_NS * _CHUNK))

    pieces = _sc_scatter(msgs, jnp.concatenate(idx_parts), tuple(counts))
    sum_msg = pieces.transpose(2, 1, 0, 3).reshape(_N_PAD, _D)

    return _update_mlp(sum_msg, node_embeddings,
                       Wu_in.T, bu_in.reshape(1, -1),
                       Wu_out.T, bu_out.reshape(1, -1))
